# baseline (speedup 1.0000x reference)
import jax
import jax.numpy as jnp
from jax import lax
from jax.experimental import pallas as pl
from jax.experimental.pallas import tpu as pltpu
from jax.experimental.pallas import tpu_sc as plsc

NC = 2
NS = 16
NW = NC * NS
L = 16

B = 4096
T = 4096
W = 2 * T
NB = 64
VOCAB = 128
ROWS_PER_W = B // NW
K = 4
NG = ROWS_PER_W // K
INNER = W // (2 * L)


def _hist_body(obs_hbm, out_hbm, buf0, buf1, hist, outa, sem0, sem1):
    c = lax.axis_index("c")
    s = lax.axis_index("s")
    wid = s * NC + c

    lanes = lax.iota(jnp.int32, L)
    lanebase = lanes * NB
    ones = jnp.full((L,), 1, jnp.int32)
    zeros = jnp.zeros((L,), jnp.int32)

    bufs = (buf0, buf1)
    sems = (sem0, sem1)

    def src(g):
        return obs_hbm.at[wid * NG + g]

    def do_row(buf, g, k):
        def zbody(j, _):
            hist[pl.ds(j * L, L)] = zeros
            return 0

        lax.fori_loop(0, NB, zbody, 0, unroll=8)

        ihi0 = lanes * 2 + k * W

        @plsc.parallel_loop(0, INNER, unroll=8, carry=(ihi0, ihi0 + 1))
        def inner(i, carry):
            ihi, ilo = carry
            hi = plsc.load_gather(buf, [ihi])
            lo = plsc.load_gather(buf, [ilo])
            addr = lanebase + (hi << 3) + lo
            plsc.addupdate_scatter(hist, [addr], ones)
            return (ihi + 2 * L, ilo + 2 * L)

        r_off = (g * K + k) * VOCAB

        def rbody(ch, _):
            acc = hist[pl.ds(ch * L, L)]
            for lane in range(1, L):
                acc = acc + hist[pl.ds(lane * NB + ch * L, L)]
            outa[pl.ds(r_off + ch * L, L)] = acc
            return 0

        lax.fori_loop(0, NB // L, rbody, 0)
        for ch in range(NB // L, VOCAB // L):
            outa[pl.ds(r_off + ch * L, L)] = zeros

    pltpu.async_copy(src(0), buf0, sem0)

    def pair_body(gp, _):
        for b in range(2):
            g = 2 * gp + b

            @pl.when(g < NG - 1)
            def _prefetch():
                pltpu.async_copy(src(g + 1), bufs[1 - b], sems[1 - b])

            pltpu.make_async_copy(src(g), bufs[b], sems[b]).wait()

            def kbody(k, _):
                do_row(bufs[b], g, k)
                return 0

            lax.fori_loop(0, K, kbody, 0)
        return 0

    lax.fori_loop(0, NG // 2, pair_body, 0)
    pltpu.sync_copy(outa, out_hbm.at[wid])


@jax.jit
def kernel(observations):
    obs = observations.reshape(NW * NG, K * W)
    mesh = plsc.VectorSubcoreMesh(
        core_axis_name="c", subcore_axis_name="s", num_cores=NC, num_subcores=NS
    )
    run = pl.kernel(
        _hist_body,
        out_type=jax.ShapeDtypeStruct((NW, ROWS_PER_W * VOCAB), jnp.int32),
        mesh=mesh,
        scratch_types=[
            pltpu.VMEM((K * W,), jnp.int32),
            pltpu.VMEM((K * W,), jnp.int32),
            pltpu.VMEM((NB * L,), jnp.int32),
            pltpu.VMEM((ROWS_PER_W * VOCAB,), jnp.int32),
            pltpu.SemaphoreType.DMA,
            pltpu.SemaphoreType.DMA,
        ],
        compiler_params=pltpu.CompilerParams(needs_layout_passes=False),
    )
    return run(obs).reshape(B, VOCAB)


# speedup vs baseline: 59.1974x; 59.1974x over previous
import jax
import jax.numpy as jnp
from jax import lax
from jax.experimental import pallas as pl
from jax.experimental.pallas import tpu as pltpu
from jax.experimental.pallas import tpu_sc as plsc

NC = 2
NS = 16
NW = NC * NS
L = 16

B = 4096
T = 4096
W = 2 * T
NB = 64
VOCAB = 128
ROWS_PER_W = B // NW
NBUF = 4
INNER = W // (2 * L)


def _hist_body(obs_hbm, out_hbm, buf0, buf1, buf2, buf3, hist, outa,
               sem0, sem1, sem2, sem3):
    c = lax.axis_index("c")
    s = lax.axis_index("s")
    wid = s * NC + c
    row0 = wid * ROWS_PER_W

    lanes = lax.iota(jnp.int32, L)
    lanebase = lanes * NB
    ones = jnp.full((L,), 1, jnp.int32)
    zeros = jnp.zeros((L,), jnp.int32)

    bufs = (buf0, buf1, buf2, buf3)
    sems = (sem0, sem1, sem2, sem3)

    def do_row(buf, r):
        def zbody(j, _):
            hist[pl.ds(j * L, L)] = zeros
            return 0

        lax.fori_loop(0, NB, zbody, 0, unroll=8)

        ihi0 = lanes * 2

        @plsc.parallel_loop(0, INNER, unroll=8, carry=(ihi0, ihi0 + 1))
        def inner(i, carry):
            ihi, ilo = carry
            hi = plsc.load_gather(buf, [ihi])
            lo = plsc.load_gather(buf, [ilo])
            addr = lanebase + (hi << 3) + lo
            plsc.addupdate_scatter(hist, [addr], ones)
            return (ihi + 2 * L, ilo + 2 * L)

        r_off = r * VOCAB

        def rbody(ch, _):
            acc = hist[pl.ds(ch * L, L)]
            for lane in range(1, L):
                acc = acc + hist[pl.ds(lane * NB + ch * L, L)]
            outa[pl.ds(r_off + ch * L, L)] = acc
            return 0

        lax.fori_loop(0, NB // L, rbody, 0)
        for ch in range(NB // L, VOCAB // L):
            outa[pl.ds(r_off + ch * L, L)] = zeros

    for i in range(NBUF - 1):
        pltpu.async_copy(obs_hbm.at[row0 + i], bufs[i], sems[i])

    def quad_body(qp, _):
        for b in range(NBUF):
            r = NBUF * qp + b
            row = row0 + r

            @pl.when(r < ROWS_PER_W - (NBUF - 1))
            def _prefetch():
                pltpu.async_copy(obs_hbm.at[row + NBUF - 1],
                                 bufs[(b + NBUF - 1) % NBUF],
                                 sems[(b + NBUF - 1) % NBUF])

            pltpu.make_async_copy(obs_hbm.at[row], bufs[b], sems[b]).wait()
            do_row(bufs[b], r)
        return 0

    lax.fori_loop(0, ROWS_PER_W // NBUF, quad_body, 0)
    pltpu.sync_copy(outa, out_hbm.at[wid])


@jax.jit
def kernel(observations):
    obs = observations.reshape(B, W)
    mesh = plsc.VectorSubcoreMesh(
        core_axis_name="c", subcore_axis_name="s", num_cores=NC, num_subcores=NS
    )
    run = pl.kernel(
        _hist_body,
        out_type=jax.ShapeDtypeStruct((NW, ROWS_PER_W * VOCAB), jnp.int32),
        mesh=mesh,
        scratch_types=[
            pltpu.VMEM((W,), jnp.int32),
            pltpu.VMEM((W,), jnp.int32),
            pltpu.VMEM((W,), jnp.int32),
            pltpu.VMEM((W,), jnp.int32),
            pltpu.VMEM((NB * L,), jnp.int32),
            pltpu.VMEM((ROWS_PER_W * VOCAB,), jnp.int32),
            pltpu.SemaphoreType.DMA,
            pltpu.SemaphoreType.DMA,
            pltpu.SemaphoreType.DMA,
            pltpu.SemaphoreType.DMA,
        ],
        compiler_params=pltpu.CompilerParams(needs_layout_passes=False),
    )
    return run(obs).reshape(B, VOCAB)
